# Initial kernel scaffold; baseline (speedup 1.0000x reference)
import functools

import jax
import jax.numpy as jnp
from jax import lax
from jax.experimental import pallas as pl
from jax.experimental.pallas import tpu as pltpu
from jax.experimental.pallas import tpu_sc as plsc

_NUM_NODES = 10000
_NUM_EDGES = 320000
_NUM_REL = 512
_D = 128
_NW = 32
_EPW = _NUM_EDGES // _NW
_C = 80
_NCHUNK = _EPW // _C

_mesh = plsc.VectorSubcoreMesh(core_axis_name="c", subcore_axis_name="s")


@functools.partial(
    pl.kernel,
    mesh=_mesh,
    out_type=jax.ShapeDtypeStruct((_NUM_EDGES,), jnp.float32),
    scratch_types=[
        pltpu.VMEM((_NUM_REL, _D), jnp.float32),
        pltpu.VMEM((_C, _D), jnp.float32),
        pltpu.VMEM((_C, _D), jnp.float32),
        pltpu.VMEM((_C,), jnp.int32),
        pltpu.VMEM((_C,), jnp.int32),
        pltpu.VMEM((_C,), jnp.int32),
        pltpu.VMEM((_C,), jnp.float32),
        pltpu.SemaphoreType.DMA,
        pltpu.SemaphoreType.DMA,
    ],
)
def _distmult_sc(z_hbm, src_hbm, dst_hbm, et_hbm, rel_hbm, out_hbm,
                 rel_v, srow, drow, sidx, didx, tidx, oc, sem_s, sem_d):
    wid = lax.axis_index("c") * 16 + lax.axis_index("s")
    base = wid * _EPW
    pltpu.sync_copy(rel_hbm, rel_v)

    def chunk_body(j, carry):
        off = base + j * _C
        pltpu.sync_copy(src_hbm.at[pl.ds(off, _C)], sidx)
        pltpu.sync_copy(dst_hbm.at[pl.ds(off, _C)], didx)
        pltpu.sync_copy(et_hbm.at[pl.ds(off, _C)], tidx)
        cp_s = pltpu.async_copy(z_hbm.at[sidx], srow, sem_s)
        cp_d = pltpu.async_copy(z_hbm.at[didx], drow, sem_d)
        cp_s.wait()
        cp_d.wait()

        def edge_body(e, c):
            t = tidx[e]
            acc = (srow[e, pl.ds(0, 16)] * rel_v[t, pl.ds(0, 16)]
                   * drow[e, pl.ds(0, 16)])
            for k in range(1, 8):
                acc = acc + (srow[e, pl.ds(16 * k, 16)]
                             * rel_v[t, pl.ds(16 * k, 16)]
                             * drow[e, pl.ds(16 * k, 16)])
            oc[e] = jnp.sum(acc)
            return c

        lax.fori_loop(0, _C, edge_body, 0, unroll=2)
        pltpu.sync_copy(oc, out_hbm.at[pl.ds(off, _C)])
        return carry

    lax.fori_loop(0, _NCHUNK, chunk_body, 0)


def kernel(z, edge_index, edge_type, rel_emb):
    src = edge_index[0].astype(jnp.int32)
    dst = edge_index[1].astype(jnp.int32)
    et = edge_type.astype(jnp.int32)
    return _distmult_sc(z, src, dst, et, rel_emb)


# speedup vs baseline: 1.1047x; 1.1047x over previous
import functools

import jax
import jax.numpy as jnp
from jax import lax
from jax.experimental import pallas as pl
from jax.experimental.pallas import tpu as pltpu
from jax.experimental.pallas import tpu_sc as plsc

_NUM_NODES = 10000
_NUM_EDGES = 320000
_NUM_REL = 512
_D = 128
_NW = 32
_EPW = _NUM_EDGES // _NW
_C = 80
_NCHUNK = _EPW // _C

_mesh = plsc.VectorSubcoreMesh(core_axis_name="c", subcore_axis_name="s")


@functools.partial(
    pl.kernel,
    mesh=_mesh,
    compiler_params=pltpu.CompilerParams(needs_layout_passes=False),
    out_type=jax.ShapeDtypeStruct((_NUM_EDGES,), jnp.float32),
    scratch_types=[
        pltpu.VMEM((_NUM_REL, _D), jnp.float32),
        pltpu.VMEM((_C, _D), jnp.float32),
        pltpu.VMEM((_C, _D), jnp.float32),
        pltpu.VMEM((_C,), jnp.int32),
        pltpu.VMEM((_C,), jnp.int32),
        pltpu.VMEM((_C,), jnp.int32),
        pltpu.VMEM((_C,), jnp.float32),
        pltpu.SemaphoreType.DMA,
        pltpu.SemaphoreType.DMA,
    ],
)
def _distmult_sc(z_hbm, src_hbm, dst_hbm, et_hbm, rel_hbm, out_hbm,
                 rel_v, srow, drow, sidx, didx, tidx, oc, sem_s, sem_d):
    wid = lax.axis_index("c") * 16 + lax.axis_index("s")
    base = wid * _EPW
    pltpu.sync_copy(rel_hbm, rel_v)

    def chunk_body(j, carry):
        off = base + j * _C
        pltpu.sync_copy(src_hbm.at[pl.ds(off, _C)], sidx)
        pltpu.sync_copy(dst_hbm.at[pl.ds(off, _C)], didx)
        pltpu.sync_copy(et_hbm.at[pl.ds(off, _C)], tidx)
        cp_s = pltpu.async_copy(z_hbm.at[sidx], srow, sem_s)
        cp_d = pltpu.async_copy(z_hbm.at[didx], drow, sem_d)
        cp_s.wait()
        cp_d.wait()

        def group_body(g, c):
            e0 = g * 16
            tvals = tidx[pl.ds(e0, 16)]
            erow = e0 + lax.iota(jnp.int32, 16)

            def d_body(d, carry):
                acc, col = carry
                sv = plsc.load_gather(srow, [erow, col])
                dv = plsc.load_gather(drow, [erow, col])
                rv = plsc.load_gather(rel_v, [tvals, col])
                return acc + sv * rv * dv, col + 1

            acc, _ = lax.fori_loop(
                0, _D, d_body,
                (jnp.zeros((16,), jnp.float32), jnp.zeros((16,), jnp.int32)),
                unroll=8)
            oc[pl.ds(e0, 16)] = acc
            return c

        lax.fori_loop(0, _C // 16, group_body, 0)
        pltpu.sync_copy(oc, out_hbm.at[pl.ds(off, _C)])
        return carry

    lax.fori_loop(0, _NCHUNK, chunk_body, 0)


def kernel(z, edge_index, edge_type, rel_emb):
    src = edge_index[0].astype(jnp.int32)
    dst = edge_index[1].astype(jnp.int32)
    et = edge_type.astype(jnp.int32)
    return _distmult_sc(z, src, dst, et, rel_emb)


# speedup vs baseline: 4.3762x; 3.9616x over previous
import functools

import jax
import jax.numpy as jnp
from jax import lax
from jax.experimental import pallas as pl
from jax.experimental.pallas import tpu as pltpu
from jax.experimental.pallas import tpu_sc as plsc

_NUM_NODES = 10000
_NUM_EDGES = 320000
_NUM_REL = 512
_D = 128
_NW = 32
_EPW = _NUM_EDGES // _NW
_C = 80
_NCHUNK = _EPW // _C

_mesh = plsc.VectorSubcoreMesh(core_axis_name="c", subcore_axis_name="s")

_GATHER_DNUMS = lax.GatherDimensionNumbers(
    offset_dims=(), collapsed_slice_dims=(0,), start_index_map=(0,))


def _permute(v, idx):
    return lax.gather(v, idx[:, None], _GATHER_DNUMS, slice_sizes=(1,),
                      mode=lax.GatherScatterMode.PROMISE_IN_BOUNDS)


@functools.partial(
    pl.kernel,
    mesh=_mesh,
    compiler_params=pltpu.CompilerParams(needs_layout_passes=False),
    out_type=jax.ShapeDtypeStruct((_NUM_EDGES,), jnp.float32),
    scratch_types=[
        pltpu.VMEM((_NUM_REL, _D), jnp.float32),
        pltpu.VMEM((2, _C, _D), jnp.float32),
        pltpu.VMEM((2, _C, _D), jnp.float32),
        pltpu.VMEM((2, _C), jnp.int32),
        pltpu.VMEM((2, _C), jnp.int32),
        pltpu.VMEM((2, _C), jnp.int32),
        pltpu.VMEM((_C,), jnp.float32),
        pltpu.SemaphoreType.DMA,
        pltpu.SemaphoreType.DMA,
    ],
)
def _distmult_sc(z_hbm, src_hbm, dst_hbm, et_hbm, rel_hbm, out_hbm,
                 rel_v, srow, drow, sidx, didx, tidx, oc, sem0, sem1):
    wid = lax.axis_index("c") * 16 + lax.axis_index("s")
    base = wid * _EPW
    pltpu.sync_copy(rel_hbm, rel_v)
    sems = (sem0, sem1)

    def issue(j, b):
        off = base + j * _C
        pltpu.sync_copy(src_hbm.at[pl.ds(off, _C)], sidx.at[b])
        pltpu.sync_copy(dst_hbm.at[pl.ds(off, _C)], didx.at[b])
        pltpu.sync_copy(et_hbm.at[pl.ds(off, _C)], tidx.at[b])
        pltpu.async_copy(z_hbm.at[sidx.at[b]], srow.at[b], sems[b])
        pltpu.async_copy(z_hbm.at[didx.at[b]], drow.at[b], sems[b])

    def process(j, b):
        pltpu.make_async_copy(z_hbm.at[sidx.at[b]], srow.at[b], sems[b]).wait()
        pltpu.make_async_copy(z_hbm.at[didx.at[b]], drow.at[b], sems[b]).wait()

        lane = lax.iota(jnp.int32, 16)

        def group_body(g, c):
            e0 = g * 16
            tvals = tidx[b, pl.ds(e0, 16)]
            res = jnp.zeros((16,), jnp.float32)
            for jj in range(16):
                e = e0 + jj
                t_spl = _permute(tvals, lane * 0 + jj)
                acc = None
                for k in range(8):
                    sv = srow[b, e, pl.ds(16 * k, 16)]
                    dv = drow[b, e, pl.ds(16 * k, 16)]
                    rv = plsc.load_gather(rel_v, [t_spl, lane + 16 * k])
                    p = sv * dv * rv
                    acc = p if acc is None else acc + p
                for m in (8, 4, 2, 1):
                    acc = acc + _permute(acc, lane ^ m)
                res = jnp.where(lane == jj, acc, res)
            oc[pl.ds(e0, 16)] = res
            return c

        lax.fori_loop(0, _C // 16, group_body, 0)
        pltpu.sync_copy(oc, out_hbm.at[pl.ds(base + j * _C, _C)])

    issue(0, 0)

    def pair_body(i, carry):
        j0 = 2 * i
        issue(j0 + 1, 1)
        process(j0, 0)
        issue(j0 + 2, 0)
        process(j0 + 1, 1)
        return carry

    lax.fori_loop(0, (_NCHUNK - 1) // 2, pair_body, 0)
    process(_NCHUNK - 1, 0)


def kernel(z, edge_index, edge_type, rel_emb):
    src = edge_index[0].astype(jnp.int32)
    dst = edge_index[1].astype(jnp.int32)
    et = edge_type.astype(jnp.int32)
    return _distmult_sc(z, src, dst, et, rel_emb)


# speedup vs baseline: 5.3731x; 1.2278x over previous
import functools

import jax
import jax.numpy as jnp
from jax import lax
from jax.experimental import pallas as pl
from jax.experimental.pallas import tpu as pltpu
from jax.experimental.pallas import tpu_sc as plsc

_NUM_NODES = 10000
_NUM_EDGES = 320000
_NUM_REL = 512
_D = 128
_NW = 32
_EPW = _NUM_EDGES // _NW
_C = 80
_NCHUNK = _EPW // _C

_mesh = plsc.VectorSubcoreMesh(core_axis_name="c", subcore_axis_name="s")

_GATHER_DNUMS = lax.GatherDimensionNumbers(
    offset_dims=(), collapsed_slice_dims=(0,), start_index_map=(0,))


def _permute(v, idx):
    return lax.gather(v, idx[:, None], _GATHER_DNUMS, slice_sizes=(1,),
                      mode=lax.GatherScatterMode.PROMISE_IN_BOUNDS)


@functools.partial(
    pl.kernel,
    mesh=_mesh,
    compiler_params=pltpu.CompilerParams(needs_layout_passes=False),
    out_type=jax.ShapeDtypeStruct((_NUM_EDGES,), jnp.float32),
    scratch_types=[
        pltpu.VMEM((_NUM_REL * _D,), jnp.float32),
        pltpu.VMEM((2, _C, _D), jnp.float32),
        pltpu.VMEM((2, _C, _D), jnp.float32),
        pltpu.VMEM((2, _C), jnp.int32),
        pltpu.VMEM((2, _C), jnp.int32),
        pltpu.VMEM((2, _C), jnp.int32),
        pltpu.VMEM((2, _C), jnp.float32),
        pltpu.SemaphoreType.DMA,
        pltpu.SemaphoreType.DMA,
        pltpu.SemaphoreType.DMA,
        pltpu.SemaphoreType.DMA,
        pltpu.SemaphoreType.DMA,
        pltpu.SemaphoreType.DMA,
    ],
)
def _distmult_sc(z_hbm, src_hbm, dst_hbm, et_hbm, rel_hbm, out_hbm,
                 rel_v, srow, drow, sidx, didx, tidx, oc,
                 sem_i0, sem_i1, sem_r0, sem_r1, sem_o0, sem_o1):
    wid = lax.axis_index("c") * 16 + lax.axis_index("s")
    base = wid * _EPW
    sem_i = (sem_i0, sem_i1)
    sem_r = (sem_r0, sem_r1)
    sem_o = (sem_o0, sem_o1)
    pltpu.sync_copy(rel_hbm, rel_v)
    lane = lax.iota(jnp.int32, 16)

    def idx_fetch(j, b):
        off = base + j * _C
        pltpu.async_copy(src_hbm.at[pl.ds(off, _C)], sidx.at[b], sem_i[b])
        pltpu.async_copy(dst_hbm.at[pl.ds(off, _C)], didx.at[b], sem_i[b])
        pltpu.async_copy(et_hbm.at[pl.ds(off, _C)], tidx.at[b], sem_i[b])

    def rows_issue(j, b):
        for ref in (sidx, didx, tidx):
            pltpu.make_async_copy(src_hbm.at[pl.ds(base, _C)],
                                  ref.at[b], sem_i[b]).wait()
        pltpu.async_copy(z_hbm.at[sidx.at[b]], srow.at[b], sem_r[b])
        pltpu.async_copy(z_hbm.at[didx.at[b]], drow.at[b], sem_r[b])

    def process(j, b, first, fetch_next):
        pltpu.make_async_copy(z_hbm.at[sidx.at[b]], srow.at[b],
                              sem_r[b]).wait()
        pltpu.make_async_copy(z_hbm.at[didx.at[b]], drow.at[b],
                              sem_r[b]).wait()
        @pl.when(jnp.logical_not(first))
        def _():
            pltpu.make_async_copy(oc.at[b], out_hbm.at[pl.ds(base, _C)],
                                  sem_o[b]).wait()

        def group_body(g, c):
            e0 = g * 16
            tvals = tidx[b, pl.ds(e0, 16)]
            res = jnp.zeros((16,), jnp.float32)
            for jj in range(16):
                e = e0 + jj
                t_spl = _permute(tvals, lane * 0 + jj)
                rbase = t_spl * _D + lane
                acc = None
                for k in range(8):
                    sv = srow[b, e, pl.ds(16 * k, 16)]
                    dv = drow[b, e, pl.ds(16 * k, 16)]
                    rv = plsc.load_gather(rel_v, [rbase + 16 * k])
                    p = sv * dv * rv
                    acc = p if acc is None else acc + p
                for m in (8, 4, 2, 1):
                    acc = acc + _permute(acc, lane ^ m)
                res = jnp.where(lane == jj, acc, res)
            oc[b, pl.ds(e0, 16)] = res
            return c

        lax.fori_loop(0, _C // 16, group_body, 0)
        if fetch_next is not None:
            fetch_next()
        pltpu.async_copy(oc.at[b], out_hbm.at[pl.ds(base + j * _C, _C)],
                         sem_o[b])

    idx_fetch(0, 0)
    rows_issue(0, 0)
    idx_fetch(1, 1)

    def pair_body(i, carry):
        j0 = 2 * i
        rows_issue(j0 + 1, 1)
        process(j0, 0, i == 0, lambda: idx_fetch(j0 + 2, 0))
        rows_issue(j0 + 2, 0)

        def fetch_next_odd():
            @pl.when(j0 + 3 < _NCHUNK)
            def _():
                idx_fetch(j0 + 3, 1)
        process(j0 + 1, 1, i == 0, fetch_next_odd)
        return carry

    lax.fori_loop(0, (_NCHUNK - 1) // 2, pair_body, 0)
    process(_NCHUNK - 1, 0, False, None)
    pltpu.make_async_copy(oc.at[0], out_hbm.at[pl.ds(base, _C)],
                          sem_o[0]).wait()
    pltpu.make_async_copy(oc.at[1], out_hbm.at[pl.ds(base, _C)],
                          sem_o[1]).wait()


def kernel(z, edge_index, edge_type, rel_emb):
    src = edge_index[0].astype(jnp.int32)
    dst = edge_index[1].astype(jnp.int32)
    et = edge_type.astype(jnp.int32)
    return _distmult_sc(z, src, dst, et, rel_emb.reshape(-1))


# speedup vs baseline: 9.4509x; 1.7589x over previous
import functools

import jax
import jax.numpy as jnp
from jax import lax
from jax.experimental import pallas as pl
from jax.experimental.pallas import tpu as pltpu
from jax.experimental.pallas import tpu_sc as plsc

_NUM_NODES = 10000
_NUM_EDGES = 320000
_NUM_REL = 512
_D = 128
_NW = 32
_EPW = _NUM_EDGES // _NW
_C = 80
_NCHUNK = _EPW // _C

_mesh = plsc.VectorSubcoreMesh(core_axis_name="c", subcore_axis_name="s")

_GATHER_DNUMS = lax.GatherDimensionNumbers(
    offset_dims=(), collapsed_slice_dims=(0,), start_index_map=(0,))


def _permute(v, idx):
    return lax.gather(v, idx[:, None], _GATHER_DNUMS, slice_sizes=(1,),
                      mode=lax.GatherScatterMode.PROMISE_IN_BOUNDS)


@functools.partial(
    pl.kernel,
    mesh=_mesh,
    compiler_params=pltpu.CompilerParams(needs_layout_passes=False,
                                         use_tc_tiling_on_sc=False),
    out_type=jax.ShapeDtypeStruct((_NUM_EDGES,), jnp.float32),
    scratch_types=[
        pltpu.VMEM((_NUM_REL * _D,), jnp.float32),
        pltpu.VMEM((2, _C, _D // 2), jnp.int32),
        pltpu.VMEM((2, _C, _D // 2), jnp.int32),
        pltpu.VMEM((2, _C), jnp.int32),
        pltpu.VMEM((2, _C), jnp.int32),
        pltpu.VMEM((2, _C), jnp.int32),
        pltpu.VMEM((2, _C), jnp.float32),
        pltpu.SemaphoreType.DMA,
        pltpu.SemaphoreType.DMA,
        pltpu.SemaphoreType.DMA,
        pltpu.SemaphoreType.DMA,
        pltpu.SemaphoreType.DMA,
        pltpu.SemaphoreType.DMA,
    ],
)
def _distmult_sc(z_hbm, src_hbm, dst_hbm, et_hbm, rel_hbm, out_hbm,
                 rel_v, srow, drow, sidx, didx, tidx, oc,
                 sem_i0, sem_i1, sem_r0, sem_r1, sem_o0, sem_o1):
    wid = lax.axis_index("c") * 16 + lax.axis_index("s")
    base = wid * _EPW
    sem_i = (sem_i0, sem_i1)
    sem_r = (sem_r0, sem_r1)
    sem_o = (sem_o0, sem_o1)
    pltpu.sync_copy(rel_hbm, rel_v)
    lane = lax.iota(jnp.int32, 16)

    def idx_fetch(j, b):
        off = base + j * _C
        pltpu.async_copy(src_hbm.at[pl.ds(off, _C)], sidx.at[b], sem_i[b])
        pltpu.async_copy(dst_hbm.at[pl.ds(off, _C)], didx.at[b], sem_i[b])
        pltpu.async_copy(et_hbm.at[pl.ds(off, _C)], tidx.at[b], sem_i[b])

    def rows_issue(j, b):
        for ref in (sidx, didx, tidx):
            pltpu.make_async_copy(src_hbm.at[pl.ds(base, _C)],
                                  ref.at[b], sem_i[b]).wait()
        pltpu.async_copy(z_hbm.at[sidx.at[b]], srow.at[b], sem_r[b])
        pltpu.async_copy(z_hbm.at[didx.at[b]], drow.at[b], sem_r[b])

    def process(j, b, first, fetch_next):
        pltpu.make_async_copy(z_hbm.at[sidx.at[b]], srow.at[b],
                              sem_r[b]).wait()
        pltpu.make_async_copy(z_hbm.at[didx.at[b]], drow.at[b],
                              sem_r[b]).wait()
        @pl.when(jnp.logical_not(first))
        def _():
            pltpu.make_async_copy(oc.at[b], out_hbm.at[pl.ds(base, _C)],
                                  sem_o[b]).wait()

        def group_body(g, c):
            e0 = g * 16
            tvals = tidx[b, pl.ds(e0, 16)]
            res = jnp.zeros((16,), jnp.float32)
            for jj in range(16):
                e = e0 + jj
                t_spl = _permute(tvals, lane * 0 + jj)
                rbase = t_spl * _D + 2 * lane
                acc = None
                for k in range(4):
                    sl = plsc.bitcast(srow[b, e, pl.ds(16 * k, 16)],
                                      jnp.bfloat16)
                    dl = plsc.bitcast(drow[b, e, pl.ds(16 * k, 16)],
                                      jnp.bfloat16)
                    s_a, s_b = plsc.unpack(
                        sl, format=plsc.PackFormat.INTERLEAVED)
                    d_a, d_b = plsc.unpack(
                        dl, format=plsc.PackFormat.INTERLEAVED)
                    r_a = plsc.load_gather(rel_v, [rbase + 32 * k])
                    r_b = plsc.load_gather(rel_v, [rbase + 32 * k + 1])
                    p = s_a * d_a * r_a + s_b * d_b * r_b
                    acc = p if acc is None else acc + p
                for m in (8, 4, 2, 1):
                    acc = acc + _permute(acc, lane ^ m)
                res = jnp.where(lane == jj, acc, res)
            oc[b, pl.ds(e0, 16)] = res
            return c

        lax.fori_loop(0, _C // 16, group_body, 0)
        if fetch_next is not None:
            fetch_next()
        pltpu.async_copy(oc.at[b], out_hbm.at[pl.ds(base + j * _C, _C)],
                         sem_o[b])

    idx_fetch(0, 0)
    rows_issue(0, 0)
    idx_fetch(1, 1)

    def pair_body(i, carry):
        j0 = 2 * i
        rows_issue(j0 + 1, 1)
        process(j0, 0, i == 0, lambda: idx_fetch(j0 + 2, 0))
        rows_issue(j0 + 2, 0)

        def fetch_next_odd():
            @pl.when(j0 + 3 < _NCHUNK)
            def _():
                idx_fetch(j0 + 3, 1)
        process(j0 + 1, 1, i == 0, fetch_next_odd)
        return carry

    lax.fori_loop(0, (_NCHUNK - 1) // 2, pair_body, 0)
    process(_NCHUNK - 1, 0, False, None)
    pltpu.make_async_copy(oc.at[0], out_hbm.at[pl.ds(base, _C)],
                          sem_o[0]).wait()
    pltpu.make_async_copy(oc.at[1], out_hbm.at[pl.ds(base, _C)],
                          sem_o[1]).wait()


def kernel(z, edge_index, edge_type, rel_emb):
    src = edge_index[0].astype(jnp.int32)
    dst = edge_index[1].astype(jnp.int32)
    et = edge_type.astype(jnp.int32)
    z32 = lax.bitcast_convert_type(
        z.astype(jnp.bfloat16).reshape(_NUM_NODES, _D // 2, 2), jnp.int32)
    return _distmult_sc(z32, src, dst, et, rel_emb.reshape(-1))


# speedup vs baseline: 11.1014x; 1.1746x over previous
import functools

import jax
import jax.numpy as jnp
from jax import lax
from jax.experimental import pallas as pl
from jax.experimental.pallas import tpu as pltpu
from jax.experimental.pallas import tpu_sc as plsc

_NUM_NODES = 10000
_NUM_EDGES = 320000
_NUM_REL = 512
_D = 128
_NW = 32
_EPW = _NUM_EDGES // _NW
_C = 80
_NCHUNK = _EPW // _C
_NBUF = 5
_AHEAD = 3

_mesh = plsc.VectorSubcoreMesh(core_axis_name="c", subcore_axis_name="s")

_GATHER_DNUMS = lax.GatherDimensionNumbers(
    offset_dims=(), collapsed_slice_dims=(0,), start_index_map=(0,))


def _permute(v, idx):
    return lax.gather(v, idx[:, None], _GATHER_DNUMS, slice_sizes=(1,),
                      mode=lax.GatherScatterMode.PROMISE_IN_BOUNDS)


@functools.partial(
    pl.kernel,
    mesh=_mesh,
    compiler_params=pltpu.CompilerParams(needs_layout_passes=False,
                                         use_tc_tiling_on_sc=False),
    out_type=jax.ShapeDtypeStruct((_NUM_EDGES,), jnp.float32),
    scratch_types=[
        pltpu.VMEM((_NUM_REL * _D,), jnp.float32),
        pltpu.VMEM((_NBUF, _C, _D // 2), jnp.int32),
        pltpu.VMEM((_NBUF, _C, _D // 2), jnp.int32),
        pltpu.VMEM((_NBUF, _C), jnp.int32),
        pltpu.VMEM((_NBUF, _C), jnp.int32),
        pltpu.VMEM((_NBUF, _C), jnp.int32),
        pltpu.VMEM((_NBUF, _C), jnp.float32),
    ] + [pltpu.SemaphoreType.DMA] * (3 * _NBUF),
)
def _distmult_sc(z_hbm, src_hbm, dst_hbm, et_hbm, rel_hbm, out_hbm,
                 rel_v, srow, drow, sidx, didx, tidx, oc, *sems):
    wid = lax.axis_index("c") * 16 + lax.axis_index("s")
    base = wid * _EPW
    sem_i = sems[:_NBUF]
    sem_r = sems[_NBUF:2 * _NBUF]
    sem_o = sems[2 * _NBUF:]
    pltpu.sync_copy(rel_hbm, rel_v)
    lane = lax.iota(jnp.int32, 16)

    def idx_fetch(j, b):
        off = base + j * _C
        pltpu.async_copy(src_hbm.at[pl.ds(off, _C)], sidx.at[b], sem_i[b])
        pltpu.async_copy(dst_hbm.at[pl.ds(off, _C)], didx.at[b], sem_i[b])
        pltpu.async_copy(et_hbm.at[pl.ds(off, _C)], tidx.at[b], sem_i[b])

    def rows_issue(j, b):
        for ref in (sidx, didx, tidx):
            pltpu.make_async_copy(src_hbm.at[pl.ds(base, _C)],
                                  ref.at[b], sem_i[b]).wait()
        pltpu.async_copy(z_hbm.at[sidx.at[b]], srow.at[b], sem_r[b])
        pltpu.async_copy(z_hbm.at[didx.at[b]], drow.at[b], sem_r[b])

    def process(j, b, first):
        pltpu.make_async_copy(z_hbm.at[sidx.at[b]], srow.at[b],
                              sem_r[b]).wait()
        pltpu.make_async_copy(z_hbm.at[didx.at[b]], drow.at[b],
                              sem_r[b]).wait()
        @pl.when(jnp.logical_not(first))
        def _():
            pltpu.make_async_copy(oc.at[b], out_hbm.at[pl.ds(base, _C)],
                                  sem_o[b]).wait()

        def group_body(g, c):
            e0 = g * 16
            tvals = tidx[b, pl.ds(e0, 16)]
            res = jnp.zeros((16,), jnp.float32)
            for jj in range(16):
                e = e0 + jj
                t_spl = _permute(tvals, lane * 0 + jj)
                rbase = t_spl * _D + 2 * lane
                acc = None
                for k in range(4):
                    sl = plsc.bitcast(srow[b, e, pl.ds(16 * k, 16)],
                                      jnp.bfloat16)
                    dl = plsc.bitcast(drow[b, e, pl.ds(16 * k, 16)],
                                      jnp.bfloat16)
                    s_a, s_b = plsc.unpack(
                        sl, format=plsc.PackFormat.INTERLEAVED)
                    d_a, d_b = plsc.unpack(
                        dl, format=plsc.PackFormat.INTERLEAVED)
                    r_a = plsc.load_gather(rel_v, [rbase + 32 * k])
                    r_b = plsc.load_gather(rel_v, [rbase + 32 * k + 1])
                    p = s_a * d_a * r_a + s_b * d_b * r_b
                    acc = p if acc is None else acc + p
                for m in (8, 4, 2, 1):
                    acc = acc + _permute(acc, lane ^ m)
                res = jnp.where(lane == jj, acc, res)
            oc[b, pl.ds(e0, 16)] = res
            return c

        lax.fori_loop(0, _C // 16, group_body, 0)
        @pl.when(j + _NBUF < _NCHUNK)
        def _():
            idx_fetch(j + _NBUF, b)
        pltpu.async_copy(oc.at[b], out_hbm.at[pl.ds(base + j * _C, _C)],
                         sem_o[b])

    for b in range(_NBUF):
        idx_fetch(b, b)
    for b in range(_AHEAD):
        rows_issue(b, b)

    def block_body(i, carry):
        j0 = _NBUF * i
        for b in range(_NBUF):
            j = j0 + b
            process(j, b, j < _NBUF)

            @pl.when(j + _AHEAD < _NCHUNK)
            def _():
                rows_issue(j + _AHEAD, (b + _AHEAD) % _NBUF)
        return carry

    lax.fori_loop(0, _NCHUNK // _NBUF, block_body, 0)
    for b in range(_NBUF):
        pltpu.make_async_copy(oc.at[b], out_hbm.at[pl.ds(base, _C)],
                              sem_o[b]).wait()


def kernel(z, edge_index, edge_type, rel_emb):
    src = edge_index[0].astype(jnp.int32)
    dst = edge_index[1].astype(jnp.int32)
    et = edge_type.astype(jnp.int32)
    z32 = lax.bitcast_convert_type(
        z.astype(jnp.bfloat16).reshape(_NUM_NODES, _D // 2, 2), jnp.int32)
    return _distmult_sc(z32, src, dst, et, rel_emb.reshape(-1))


# speedup vs baseline: 11.2212x; 1.0108x over previous
import functools

import jax
import jax.numpy as jnp
from jax import lax
from jax.experimental import pallas as pl
from jax.experimental.pallas import tpu as pltpu
from jax.experimental.pallas import tpu_sc as plsc

_NUM_NODES = 10000
_NUM_EDGES = 320000
_NUM_REL = 512
_D = 128
_NW = 32
_EPW = _NUM_EDGES // _NW
_C = 80
_NCHUNK = _EPW // _C
_NBUF = 5
_AHEAD = 3

_mesh = plsc.VectorSubcoreMesh(core_axis_name="c", subcore_axis_name="s")

_GATHER_DNUMS = lax.GatherDimensionNumbers(
    offset_dims=(), collapsed_slice_dims=(0,), start_index_map=(0,))


def _permute(v, idx):
    return lax.gather(v, idx[:, None], _GATHER_DNUMS, slice_sizes=(1,),
                      mode=lax.GatherScatterMode.PROMISE_IN_BOUNDS)


@functools.partial(
    pl.kernel,
    mesh=_mesh,
    compiler_params=pltpu.CompilerParams(needs_layout_passes=False,
                                         use_tc_tiling_on_sc=False),
    out_type=jax.ShapeDtypeStruct((_NUM_EDGES,), jnp.float32),
    scratch_types=[
        pltpu.VMEM((_NUM_REL * _D // 2,), jnp.int32),
        pltpu.VMEM_SHARED((_NUM_NODES, _D // 2), jnp.int32),
        pltpu.VMEM((_NBUF, _C, _D // 2), jnp.int32),
        pltpu.VMEM((_NBUF, _C, _D // 2), jnp.int32),
        pltpu.VMEM((_NBUF, _C), jnp.int32),
        pltpu.VMEM((_NBUF, _C), jnp.int32),
        pltpu.VMEM((_NBUF, _C), jnp.int32),
        pltpu.VMEM((_NBUF, _C), jnp.float32),
    ] + [pltpu.SemaphoreType.DMA] * (3 * _NBUF),
)
def _distmult_sc(z_hbm, src_hbm, dst_hbm, et_hbm, rel_hbm, out_hbm,
                 rel_v, z_sp, srow, drow, sidx, didx, tidx, oc, *sems):
    wid = lax.axis_index("c") * 16 + lax.axis_index("s")
    base = wid * _EPW
    sem_i = sems[:_NBUF]
    sem_r = sems[_NBUF:2 * _NBUF]
    sem_o = sems[2 * _NBUF:]
    @pl.when(lax.axis_index("s") == 0)
    def _():
        pltpu.sync_copy(z_hbm, z_sp)
    pltpu.sync_copy(rel_hbm, rel_v)
    plsc.subcore_barrier()
    lane = lax.iota(jnp.int32, 16)

    def idx_fetch(j, b):
        off = base + j * _C
        pltpu.async_copy(src_hbm.at[pl.ds(off, _C)], sidx.at[b], sem_i[b])
        pltpu.async_copy(dst_hbm.at[pl.ds(off, _C)], didx.at[b], sem_i[b])
        pltpu.async_copy(et_hbm.at[pl.ds(off, _C)], tidx.at[b], sem_i[b])

    def rows_issue(j, b):
        for ref in (sidx, didx, tidx):
            pltpu.make_async_copy(src_hbm.at[pl.ds(base, _C)],
                                  ref.at[b], sem_i[b]).wait()
        pltpu.async_copy(z_sp.at[sidx.at[b]], srow.at[b], sem_r[b])
        pltpu.async_copy(z_sp.at[didx.at[b]], drow.at[b], sem_r[b])

    def process(j, b, first):
        pltpu.make_async_copy(z_sp.at[sidx.at[b]], srow.at[b],
                              sem_r[b]).wait()
        pltpu.make_async_copy(z_sp.at[didx.at[b]], drow.at[b],
                              sem_r[b]).wait()
        @pl.when(jnp.logical_not(first))
        def _():
            pltpu.make_async_copy(oc.at[b], out_hbm.at[pl.ds(base, _C)],
                                  sem_o[b]).wait()

        def group_body(g, c):
            e0 = g * 16
            tvals = tidx[b, pl.ds(e0, 16)]
            res = jnp.zeros((16,), jnp.float32)
            for jj in range(16):
                e = e0 + jj
                t_spl = _permute(tvals, lane * 0 + jj)
                rbase = t_spl * (_D // 2) + lane
                acc = None
                for k in range(4):
                    sl = plsc.bitcast(srow[b, e, pl.ds(16 * k, 16)],
                                      jnp.bfloat16)
                    dl = plsc.bitcast(drow[b, e, pl.ds(16 * k, 16)],
                                      jnp.bfloat16)
                    rl = plsc.bitcast(
                        plsc.load_gather(rel_v, [rbase + 16 * k]),
                        jnp.bfloat16)
                    s_a, s_b = plsc.unpack(
                        sl, format=plsc.PackFormat.INTERLEAVED)
                    d_a, d_b = plsc.unpack(
                        dl, format=plsc.PackFormat.INTERLEAVED)
                    r_a, r_b = plsc.unpack(
                        rl, format=plsc.PackFormat.INTERLEAVED)
                    p = s_a * d_a * r_a + s_b * d_b * r_b
                    acc = p if acc is None else acc + p
                for m in (8, 4, 2, 1):
                    acc = acc + _permute(acc, lane ^ m)
                res = jnp.where(lane == jj, acc, res)
            oc[b, pl.ds(e0, 16)] = res
            return c

        lax.fori_loop(0, _C // 16, group_body, 0)
        @pl.when(j + _NBUF < _NCHUNK)
        def _():
            idx_fetch(j + _NBUF, b)
        pltpu.async_copy(oc.at[b], out_hbm.at[pl.ds(base + j * _C, _C)],
                         sem_o[b])

    for b in range(_NBUF):
        idx_fetch(b, b)
    for b in range(_AHEAD):
        rows_issue(b, b)

    def block_body(i, carry):
        j0 = _NBUF * i
        for b in range(_NBUF):
            j = j0 + b
            process(j, b, j < _NBUF)

            @pl.when(j + _AHEAD < _NCHUNK)
            def _():
                rows_issue(j + _AHEAD, (b + _AHEAD) % _NBUF)
        return carry

    lax.fori_loop(0, _NCHUNK // _NBUF, block_body, 0)
    for b in range(_NBUF):
        pltpu.make_async_copy(oc.at[b], out_hbm.at[pl.ds(base, _C)],
                              sem_o[b]).wait()


def kernel(z, edge_index, edge_type, rel_emb):
    src = edge_index[0].astype(jnp.int32)
    dst = edge_index[1].astype(jnp.int32)
    et = edge_type.astype(jnp.int32)
    z32 = lax.bitcast_convert_type(
        z.astype(jnp.bfloat16).reshape(_NUM_NODES, _D // 2, 2), jnp.int32)
    rel32 = lax.bitcast_convert_type(
        rel_emb.astype(jnp.bfloat16).reshape(_NUM_REL, _D // 2, 2),
        jnp.int32).reshape(-1)
    return _distmult_sc(z32, src, dst, et, rel32)
